# baseline (device time: 79050 ns/iter reference)
import functools

import jax
import jax.numpy as jnp
from jax import lax
from jax.experimental import pallas as pl
from jax.experimental.pallas import tpu as pltpu

N_DEV = 8
N_TOK = 2048
D_MODEL = 512
D_FF = 1024
E_LOCAL = 4
CHUNK = N_TOK // N_DEV

_sem_signal = getattr(pl, "semaphore_signal", None) or pltpu.semaphore_signal
_sem_wait = getattr(pl, "semaphore_wait", None) or pltpu.semaphore_wait
_CompilerParams = getattr(pltpu, "CompilerParams", None) or pltpu.TPUCompilerParams


def kernel(x, router_W, route_idx, expert_W, shared_W):
    def body(x_ref, router_W_ref, route_idx_ref, expert_W_ref, shared_W_ref,
             out_ref, partial_ref, send_buf, recv_bufs, send_sems, recv_sems,
             done_sem):
        my_pos = lax.axis_index("i")
        left = lax.rem(my_pos + (N_DEV - 1), N_DEV)
        right = lax.rem(my_pos + 1, N_DEV)

        barrier_sem = pltpu.get_barrier_semaphore()
        for nbr in (left, right):
            _sem_signal(barrier_sem, inc=1, device_id=(nbr,),
                        device_id_type=pl.DeviceIdType.MESH)
        _sem_wait(barrier_sem, 2)

        xf = x_ref[...]
        scores = jnp.dot(xf, router_W_ref[...], preferred_element_type=jnp.float32)
        scores = scores - jnp.max(scores, axis=-1, keepdims=True)
        e_scores = jnp.exp(scores)
        probs = e_scores / jnp.sum(e_scores, axis=-1, keepdims=True)
        ridx = route_idx_ref[...]
        eids = lax.broadcasted_iota(jnp.int32, (N_TOK, 32), 1)
        p_sel = jnp.sum(jnp.where(eids == ridx, probs, 0.0), axis=-1,
                        keepdims=True)

        base = my_pos * E_LOCAL
        acc = jnp.zeros((N_TOK, D_FF), dtype=jnp.float32)
        for k in range(E_LOCAL):
            w = jnp.where(ridx == base + k, p_sel, 0.0)
            xw = (xf * w).astype(jnp.bfloat16)
            acc = acc + jnp.dot(xw, expert_W_ref[k].astype(jnp.bfloat16),
                                preferred_element_type=jnp.float32)
        partial_ref[...] = acc.astype(jnp.bfloat16)

        x_own = x_ref[pl.ds(my_pos * CHUNK, CHUNK), :]
        shared_out = jnp.dot(x_own.astype(jnp.bfloat16),
                             shared_W_ref[...].astype(jnp.bfloat16),
                             preferred_element_type=jnp.float32)

        for s in range(N_DEV - 1):
            c_send = lax.rem(my_pos + (N_DEV - 1 - s), N_DEV)
            own = partial_ref[pl.ds(c_send * CHUNK, CHUNK), :]
            if s == 0:
                send_buf[...] = own
            else:
                send_buf[...] = recv_bufs[s - 1] + own
            rdma = pltpu.make_async_remote_copy(
                src_ref=send_buf,
                dst_ref=recv_bufs.at[s],
                send_sem=send_sems.at[s],
                recv_sem=recv_sems.at[s],
                device_id=(right,),
                device_id_type=pl.DeviceIdType.MESH,
            )
            rdma.start()
            rdma.wait()

        final = (recv_bufs[N_DEV - 2] +
                 partial_ref[pl.ds(my_pos * CHUNK, CHUNK), :])
        out_ref[...] = shared_out + final.astype(jnp.float32)

        _sem_signal(done_sem, inc=1, device_id=(left,),
                    device_id_type=pl.DeviceIdType.MESH)
        _sem_wait(done_sem, 1)

    return pl.pallas_call(
        body,
        out_shape=jax.ShapeDtypeStruct((CHUNK, D_FF), jnp.float32),
        in_specs=[pl.BlockSpec(memory_space=pltpu.VMEM)] * 5,
        out_specs=pl.BlockSpec(memory_space=pltpu.VMEM),
        scratch_shapes=[
            pltpu.VMEM((N_TOK, D_FF), jnp.bfloat16),
            pltpu.VMEM((CHUNK, D_FF), jnp.bfloat16),
            pltpu.VMEM((N_DEV - 1, CHUNK, D_FF), jnp.bfloat16),
            pltpu.SemaphoreType.DMA((N_DEV - 1,)),
            pltpu.SemaphoreType.DMA((N_DEV - 1,)),
            pltpu.SemaphoreType.REGULAR,
        ],
        compiler_params=_CompilerParams(collective_id=0),
    )(x, router_W, route_idx, expert_W, shared_W)


# device time: 52742 ns/iter; 1.4988x vs baseline; 1.4988x over previous
import jax
import jax.numpy as jnp
from jax import lax
from jax.experimental import pallas as pl
from jax.experimental.pallas import tpu as pltpu

N_DEV = 8
N_TOK = 2048
D_MODEL = 512
D_FF = 1024
E_LOCAL = 4
CHUNK = N_TOK // N_DEV

_sem_signal = getattr(pl, "semaphore_signal", None) or pltpu.semaphore_signal
_sem_wait = getattr(pl, "semaphore_wait", None) or pltpu.semaphore_wait
_CompilerParams = getattr(pltpu, "CompilerParams", None) or pltpu.TPUCompilerParams


def kernel(x, router_W, route_idx, expert_W, shared_W):
    def body(x_ref, router_W_ref, route_idx_ref, expert_W_ref, shared_W_ref,
             out_ref, p_sel_ref, send_bufs, recv_bufs, send_sems, recv_sems,
             done_sem):
        my_pos = lax.axis_index("i")

        barrier_sem = pltpu.get_barrier_semaphore()
        for off in range(1, N_DEV):
            peer = lax.rem(my_pos + off, N_DEV)
            _sem_signal(barrier_sem, inc=1, device_id=(peer,),
                        device_id_type=pl.DeviceIdType.MESH)
        _sem_wait(barrier_sem, N_DEV - 1)

        xf = x_ref[...]
        scores = jnp.dot(xf, router_W_ref[...], preferred_element_type=jnp.float32)
        scores = scores - jnp.max(scores, axis=-1, keepdims=True)
        e_scores = jnp.exp(scores)
        probs = e_scores / jnp.sum(e_scores, axis=-1, keepdims=True)
        ridx = route_idx_ref[...]
        eids = lax.broadcasted_iota(jnp.int32, (N_TOK, 32), 1)
        p_sel_ref[...] = jnp.sum(jnp.where(eids == ridx, probs, 0.0), axis=-1,
                                 keepdims=True)

        base = my_pos * E_LOCAL
        w_bf = [expert_W_ref[k].astype(jnp.bfloat16) for k in range(E_LOCAL)]

        def chunk_partial(c):
            rows = pl.ds(c * CHUNK, CHUNK)
            xc = x_ref[rows, :]
            pc = p_sel_ref[rows, :]
            rc = route_idx_ref[rows, :]
            acc = jnp.zeros((CHUNK, D_FF), dtype=jnp.float32)
            for k in range(E_LOCAL):
                w = jnp.where(rc == base + k, pc, 0.0)
                xw = (xc * w).astype(jnp.bfloat16)
                acc = acc + jnp.dot(xw, w_bf[k],
                                    preferred_element_type=jnp.float32)
            return acc

        sends = []
        for j in range(N_DEV - 1):
            c = lax.rem(my_pos + 1 + j, N_DEV)
            send_bufs[j] = chunk_partial(c).astype(jnp.bfloat16)
            rdma = pltpu.make_async_remote_copy(
                src_ref=send_bufs.at[j],
                dst_ref=recv_bufs.at[N_DEV - 2 - j],
                send_sem=send_sems.at[j],
                recv_sem=recv_sems.at[N_DEV - 2 - j],
                device_id=(c,),
                device_id_type=pl.DeviceIdType.MESH,
            )
            rdma.start()
            sends.append(rdma)

        acc = chunk_partial(my_pos)
        x_own = x_ref[pl.ds(my_pos * CHUNK, CHUNK), :]
        acc = acc + jnp.dot(x_own.astype(jnp.bfloat16),
                            shared_W_ref[...].astype(jnp.bfloat16),
                            preferred_element_type=jnp.float32)

        for slot in reversed(range(N_DEV - 1)):
            recv = pltpu.make_async_remote_copy(
                src_ref=send_bufs.at[0],
                dst_ref=recv_bufs.at[slot],
                send_sem=send_sems.at[0],
                recv_sem=recv_sems.at[slot],
                device_id=(my_pos,),
                device_id_type=pl.DeviceIdType.MESH,
            )
            recv.wait_recv()
            acc = acc + recv_bufs[slot].astype(jnp.float32)

        out_ref[...] = acc

        for rdma in sends:
            rdma.wait_send()

        for off in range(1, N_DEV):
            peer = lax.rem(my_pos + off, N_DEV)
            _sem_signal(done_sem, inc=1, device_id=(peer,),
                        device_id_type=pl.DeviceIdType.MESH)
        _sem_wait(done_sem, N_DEV - 1)

    return pl.pallas_call(
        body,
        out_shape=jax.ShapeDtypeStruct((CHUNK, D_FF), jnp.float32),
        in_specs=[pl.BlockSpec(memory_space=pltpu.VMEM)] * 5,
        out_specs=pl.BlockSpec(memory_space=pltpu.VMEM),
        scratch_shapes=[
            pltpu.VMEM((N_TOK, 1), jnp.float32),
            pltpu.VMEM((N_DEV - 1, CHUNK, D_FF), jnp.bfloat16),
            pltpu.VMEM((N_DEV - 1, CHUNK, D_FF), jnp.bfloat16),
            pltpu.SemaphoreType.DMA((N_DEV - 1,)),
            pltpu.SemaphoreType.DMA((N_DEV - 1,)),
            pltpu.SemaphoreType.REGULAR,
        ],
        compiler_params=_CompilerParams(collective_id=0),
    )(x, router_W, route_idx, expert_W, shared_W)


# device time: 43538 ns/iter; 1.8157x vs baseline; 1.2114x over previous
import jax
import jax.numpy as jnp
from jax import lax
from jax.experimental import pallas as pl
from jax.experimental.pallas import tpu as pltpu

N_DEV = 8
N_TOK = 2048
D_MODEL = 512
D_FF = 1024
E_LOCAL = 4
CHUNK = N_TOK // N_DEV
G = 32
N_G = CHUNK // G

_sem_signal = getattr(pl, "semaphore_signal", None) or pltpu.semaphore_signal
_sem_wait = getattr(pl, "semaphore_wait", None) or pltpu.semaphore_wait
_CompilerParams = getattr(pltpu, "CompilerParams", None) or pltpu.TPUCompilerParams


def kernel(x, router_W, route_idx, expert_W, shared_W):
    def body(x_ref, router_W_ref, route_idx_ref, expert_W_ref, shared_W_ref,
             out_ref, p_sel_ref, send_bufs, recv_bufs, send_sems, recv_sems,
             done_sem):
        my_pos = lax.axis_index("i")

        barrier_sem = pltpu.get_barrier_semaphore()
        for off in range(1, N_DEV):
            peer = lax.rem(my_pos + off, N_DEV)
            _sem_signal(barrier_sem, inc=1, device_id=(peer,),
                        device_id_type=pl.DeviceIdType.MESH)
        _sem_wait(barrier_sem, N_DEV - 1)

        xf = x_ref[...]
        scores = jnp.dot(xf, router_W_ref[...], preferred_element_type=jnp.float32)
        scores = scores - jnp.max(scores, axis=-1, keepdims=True)
        e_scores = jnp.exp(scores)
        probs = e_scores / jnp.sum(e_scores, axis=-1, keepdims=True)
        ridx = route_idx_ref[...]
        eids = lax.broadcasted_iota(jnp.int32, (N_TOK, 32), 1)
        p_sel_ref[...] = jnp.sum(jnp.where(eids == ridx, probs, 0.0), axis=-1,
                                 keepdims=True)

        base = my_pos * E_LOCAL
        w_bf = [expert_W_ref[k].astype(jnp.bfloat16) for k in range(E_LOCAL)]

        tri_r = lax.broadcasted_iota(jnp.int32, (CHUNK, CHUNK), 0)
        tri_c = lax.broadcasted_iota(jnp.int32, (CHUNK, CHUNK), 1)
        L = (tri_c <= tri_r).astype(jnp.float32)

        def pack_matrix(chunk_pos, dev):
            rc = route_idx_ref[pl.ds(chunk_pos * CHUNK, CHUNK), :]
            b = dev * E_LOCAL
            m = ((rc >= b) & (rc < b + E_LOCAL)).astype(jnp.float32)
            rank = jnp.dot(L, m, preferred_element_type=jnp.float32)
            count = jnp.sum(m)
            rank_i = rank.astype(jnp.int32)
            q = jnp.where((rank_i - 1 == tri_c) & (m > 0.0), 1.0, 0.0)
            return count, q

        def chunk_partial(c):
            rows = pl.ds(c * CHUNK, CHUNK)
            xc = x_ref[rows, :]
            pc = p_sel_ref[rows, :]
            rc = route_idx_ref[rows, :]
            acc = jnp.zeros((CHUNK, D_FF), dtype=jnp.float32)
            for k in range(E_LOCAL):
                w = jnp.where(rc == base + k, pc, 0.0)
                xw = (xc * w).astype(jnp.bfloat16)
                acc = acc + jnp.dot(xw, w_bf[k],
                                    preferred_element_type=jnp.float32)
            return acc

        sends = []
        for j in range(N_DEV - 1):
            c = lax.rem(my_pos + 1 + j, N_DEV)
            count, q = pack_matrix(c, my_pos)
            partial = chunk_partial(c)
            packed = lax.dot_general(
                q, partial, (((0,), (0,)), ((), ())),
                preferred_element_type=jnp.float32)
            send_bufs[j] = packed.astype(jnp.bfloat16).reshape(N_G, G, D_FF)
            slot = N_DEV - 2 - j
            for g in range(N_G):
                rdma = pltpu.make_async_remote_copy(
                    src_ref=send_bufs.at[j, g],
                    dst_ref=recv_bufs.at[slot, g],
                    send_sem=send_sems.at[j, g],
                    recv_sem=recv_sems.at[slot, g],
                    device_id=(c,),
                    device_id_type=pl.DeviceIdType.MESH,
                )

                @pl.when(count > g * G)
                def _(rdma=rdma):
                    rdma.start()

                sends.append((count, g, rdma))

        acc = chunk_partial(my_pos)
        x_own = x_ref[pl.ds(my_pos * CHUNK, CHUNK), :]
        acc = acc + jnp.dot(x_own.astype(jnp.bfloat16),
                            shared_W_ref[...].astype(jnp.bfloat16),
                            preferred_element_type=jnp.float32)

        for slot in reversed(range(N_DEV - 1)):
            sender = lax.rem(my_pos + slot + 1, N_DEV)
            count, q = pack_matrix(my_pos, sender)
            for g in range(N_G):
                recv = pltpu.make_async_remote_copy(
                    src_ref=send_bufs.at[0, 0],
                    dst_ref=recv_bufs.at[slot, g],
                    send_sem=send_sems.at[0, 0],
                    recv_sem=recv_sems.at[slot, g],
                    device_id=(my_pos,),
                    device_id_type=pl.DeviceIdType.MESH,
                )

                @pl.when(count > g * G)
                def _(recv=recv):
                    recv.wait_recv()

            row_ids = lax.broadcasted_iota(jnp.int32, (CHUNK, 1), 0)
            packed = recv_bufs[slot].reshape(CHUNK, D_FF).astype(jnp.float32)
            packed = jnp.where(row_ids < count.astype(jnp.int32), packed, 0.0)
            acc = acc + jnp.dot(q, packed, preferred_element_type=jnp.float32)

        out_ref[...] = acc

        for count, g, rdma in sends:
            @pl.when(count > g * G)
            def _(rdma=rdma):
                rdma.wait_send()

        for off in range(1, N_DEV):
            peer = lax.rem(my_pos + off, N_DEV)
            _sem_signal(done_sem, inc=1, device_id=(peer,),
                        device_id_type=pl.DeviceIdType.MESH)
        _sem_wait(done_sem, N_DEV - 1)

    return pl.pallas_call(
        body,
        out_shape=jax.ShapeDtypeStruct((CHUNK, D_FF), jnp.float32),
        in_specs=[pl.BlockSpec(memory_space=pltpu.VMEM)] * 5,
        out_specs=pl.BlockSpec(memory_space=pltpu.VMEM),
        scratch_shapes=[
            pltpu.VMEM((N_TOK, 1), jnp.float32),
            pltpu.VMEM((N_DEV - 1, N_G, G, D_FF), jnp.bfloat16),
            pltpu.VMEM((N_DEV - 1, N_G, G, D_FF), jnp.bfloat16),
            pltpu.SemaphoreType.DMA((N_DEV - 1, N_G)),
            pltpu.SemaphoreType.DMA((N_DEV - 1, N_G)),
            pltpu.SemaphoreType.REGULAR,
        ],
        compiler_params=_CompilerParams(collective_id=0),
    )(x, router_W, route_idx, expert_W, shared_W)
